# baseline (speedup 1.0000x reference)
import numpy as np
import jax
import jax.numpy as jnp
from jax import lax
from jax.experimental import pallas as pl
from jax.experimental.pallas import tpu as pltpu

_H_PAD = 80


def _ceil_to(x: int, m: int) -> int:
    return ((x + m - 1) // m) * m


def _mlp_body(y_ref, w1_ref, b1_ref, w2_ref, b2_ref, o_ref):
    h = jnp.maximum(w1_ref[...] * y_ref[...] + b1_ref[...], 0.0)
    o_ref[...] = (
        jnp.dot(w2_ref[...], h, preferred_element_type=jnp.float32) + b2_ref[...]
    )


def _priors_flat(y_flat, w1, b1, w2, b2, *, tile_n=32768):
    H = w1.shape[1]
    S = w2.shape[1]
    pad_h = _H_PAD - H
    w1c = jnp.pad(jnp.asarray(w1, jnp.float32).reshape(H, 1), ((0, pad_h), (0, 0)))
    b1c = jnp.pad(jnp.asarray(b1, jnp.float32).reshape(H, 1), ((0, pad_h), (0, 0)))
    w2t = jnp.pad(jnp.asarray(w2, jnp.float32).T, ((0, 0), (0, pad_h)))
    b2c = jnp.asarray(b2, jnp.float32).reshape(S, 1)

    N = y_flat.shape[1]
    Np = _ceil_to(N, tile_n)
    if Np != N:
        y_flat = jnp.pad(y_flat, ((0, 0), (0, Np - N)))

    return pl.pallas_call(
        _mlp_body,
        out_shape=jax.ShapeDtypeStruct((S, Np), jnp.float32),
        grid=(Np // tile_n,),
        in_specs=[
            pl.BlockSpec((1, tile_n), lambda i: (0, i)),
            pl.BlockSpec((_H_PAD, 1), lambda i: (0, 0)),
            pl.BlockSpec((_H_PAD, 1), lambda i: (0, 0)),
            pl.BlockSpec((S, _H_PAD), lambda i: (0, 0)),
            pl.BlockSpec((S, 1), lambda i: (0, 0)),
        ],
        out_specs=pl.BlockSpec((S, tile_n), lambda i: (0, i)),
        compiler_params=pltpu.CompilerParams(dimension_semantics=("parallel",)),
    )(y_flat, w1c, b1c, w2t, b2c)[:, :N]


def _make_viterbi_body(tile_t: int, unroll: int):
    def body(p_ref, det_ref, pt_ref, dt_ref, uv_ref):
        @pl.when(pl.program_id(0) == 0)
        def _init():
            uv_ref[...] = jnp.zeros_like(uv_ref)

        pt_ref[...] = jnp.transpose(p_ref[...], (0, 2, 1))

        def step(i, carry):
            u, v = carry
            dt_ref[pl.ds(i, 1), :] = jnp.where(u <= v, 0.0, 1.0)
            pt = pt_ref[:, i, :]
            u2 = jnp.minimum(u - pt[0:1], v - pt[1:2])
            v2 = jnp.minimum(u - pt[2:3], v - pt[3:4])
            return (u2, v2)

        u0 = uv_ref[0:1, :]
        v0 = uv_ref[1:2, :]
        uf, vf = lax.fori_loop(0, tile_t, step, (u0, v0), unroll=unroll)
        uv_ref[0:1, :] = uf
        uv_ref[1:2, :] = vf

        det_ref[...] = jnp.transpose(dt_ref[...], (1, 0))

    return body


def _viterbi_bits(priors_sbt, *, tile_t=2048, unroll=16):
    S, B, T = priors_sbt.shape
    tile_t = int(min(tile_t, _ceil_to(T, 8)))
    Tp = _ceil_to(T, tile_t)
    if Tp != T:
        priors_sbt = jnp.pad(priors_sbt, ((0, 0), (0, 0), (0, Tp - T)))

    det = pl.pallas_call(
        _make_viterbi_body(tile_t, int(min(unroll, tile_t))),
        out_shape=jax.ShapeDtypeStruct((B, Tp), jnp.float32),
        grid=(Tp // tile_t,),
        in_specs=[pl.BlockSpec((S, B, tile_t), lambda t: (0, 0, t))],
        out_specs=pl.BlockSpec((B, tile_t), lambda t: (0, t)),
        scratch_shapes=[
            pltpu.VMEM((S, tile_t, B), jnp.float32),
            pltpu.VMEM((tile_t, B), jnp.float32),
            pltpu.VMEM((2, B), jnp.float32),
        ],
        compiler_params=pltpu.CompilerParams(dimension_semantics=("arbitrary",)),
    )(priors_sbt)
    return det[:, :T]


def kernel(y, w1, b1, w2, b2):
    B, T = y.shape
    S = w2.shape[1]
    y_flat = y.reshape(1, B * T).astype(jnp.float32)
    priors = _priors_flat(y_flat, w1, b1, w2, b2)
    return _viterbi_bits(priors.reshape(S, B, T))


# speedup vs baseline: 1.0636x; 1.0636x over previous
import numpy as np
import jax
import jax.numpy as jnp
from jax import lax
from jax.experimental import pallas as pl
from jax.experimental.pallas import tpu as pltpu

_H_PAD = 80


def _ceil_to(x: int, m: int) -> int:
    return ((x + m - 1) // m) * m


def _mlp_body(y_ref, w1_ref, b1_ref, w2_ref, b2_ref, o_ref):
    h = jnp.maximum(w1_ref[...] * y_ref[...] + b1_ref[...], 0.0)
    o_ref[...] = (
        jnp.dot(w2_ref[...], h, preferred_element_type=jnp.float32) + b2_ref[...]
    )


def _priors_flat(y_flat, w1, b1, w2, b2, *, tile_n=32768):
    H = w1.shape[1]
    S = w2.shape[1]
    pad_h = _H_PAD - H
    w1c = jnp.pad(jnp.asarray(w1, jnp.float32).reshape(H, 1), ((0, pad_h), (0, 0)))
    b1c = jnp.pad(jnp.asarray(b1, jnp.float32).reshape(H, 1), ((0, pad_h), (0, 0)))
    w2t = jnp.pad(jnp.asarray(w2, jnp.float32).T, ((0, 0), (0, pad_h)))
    b2c = jnp.asarray(b2, jnp.float32).reshape(S, 1)

    N = y_flat.shape[1]
    Np = _ceil_to(N, tile_n)
    if Np != N:
        y_flat = jnp.pad(y_flat, ((0, 0), (0, Np - N)))

    return pl.pallas_call(
        _mlp_body,
        out_shape=jax.ShapeDtypeStruct((S, Np), jnp.float32),
        grid=(Np // tile_n,),
        in_specs=[
            pl.BlockSpec((1, tile_n), lambda i: (0, i)),
            pl.BlockSpec((_H_PAD, 1), lambda i: (0, 0)),
            pl.BlockSpec((_H_PAD, 1), lambda i: (0, 0)),
            pl.BlockSpec((S, _H_PAD), lambda i: (0, 0)),
            pl.BlockSpec((S, 1), lambda i: (0, 0)),
        ],
        out_specs=pl.BlockSpec((S, tile_n), lambda i: (0, i)),
        compiler_params=pltpu.CompilerParams(dimension_semantics=("parallel",)),
    )(y_flat, w1c, b1c, w2t, b2c)[:, :N]


def _make_viterbi_body(tile_t: int, unroll: int):
    def body(p_ref, det_ref, dt_ref, uv_ref):
        @pl.when(pl.program_id(0) == 0)
        def _init():
            uv_ref[...] = jnp.zeros_like(uv_ref)

        def step(i, carry):
            u, v = carry
            dt_ref[pl.ds(i, 1), :] = jnp.where(u <= v, 0.0, 1.0)
            pt = p_ref[:, i, :]
            u2 = jnp.minimum(u - pt[0:1], v - pt[1:2])
            v2 = jnp.minimum(u - pt[2:3], v - pt[3:4])
            return (u2, v2)

        u0 = uv_ref[0:1, :]
        v0 = uv_ref[1:2, :]
        uf, vf = lax.fori_loop(0, tile_t, step, (u0, v0), unroll=unroll)
        uv_ref[0:1, :] = uf
        uv_ref[1:2, :] = vf

        det_ref[...] = jnp.transpose(dt_ref[...], (1, 0))

    return body


def _viterbi_bits(priors_stb, *, tile_t=2048, unroll=16):
    S, T, B = priors_stb.shape
    tile_t = int(min(tile_t, _ceil_to(T, 8)))
    Tp = _ceil_to(T, tile_t)
    if Tp != T:
        priors_stb = jnp.pad(priors_stb, ((0, 0), (0, Tp - T), (0, 0)))

    det = pl.pallas_call(
        _make_viterbi_body(tile_t, int(min(unroll, tile_t))),
        out_shape=jax.ShapeDtypeStruct((B, Tp), jnp.float32),
        grid=(Tp // tile_t,),
        in_specs=[pl.BlockSpec((S, tile_t, B), lambda t: (0, t, 0))],
        out_specs=pl.BlockSpec((B, tile_t), lambda t: (0, t)),
        scratch_shapes=[
            pltpu.VMEM((tile_t, B), jnp.float32),
            pltpu.VMEM((2, B), jnp.float32),
        ],
        compiler_params=pltpu.CompilerParams(dimension_semantics=("arbitrary",)),
    )(priors_stb)
    return det[:, :T]


def kernel(y, w1, b1, w2, b2):
    B, T = y.shape
    S = w2.shape[1]
    y_flat = y.T.reshape(1, T * B).astype(jnp.float32)
    priors = _priors_flat(y_flat, w1, b1, w2, b2)
    return _viterbi_bits(priors.reshape(S, T, B))


# speedup vs baseline: 1.1037x; 1.0377x over previous
import numpy as np
import jax
import jax.numpy as jnp
from jax import lax
from jax.experimental import pallas as pl
from jax.experimental.pallas import tpu as pltpu

_H_PAD = 80


def _ceil_to(x: int, m: int) -> int:
    return ((x + m - 1) // m) * m


def _mlp_body(y_ref, w1_ref, b1_ref, w2_ref, b2_ref, o_ref):
    h = jnp.maximum(w1_ref[...] * y_ref[...] + b1_ref[...], 0.0)
    o_ref[...] = (
        jnp.dot(w2_ref[...], h, preferred_element_type=jnp.float32) + b2_ref[...]
    )


def _priors_flat(y_flat, w1, b1, w2, b2, *, tile_n=65536):
    H = w1.shape[1]
    S = w2.shape[1]
    pad_h = _H_PAD - H
    w1c = jnp.pad(jnp.asarray(w1, jnp.float32).reshape(H, 1), ((0, pad_h), (0, 0)))
    b1c = jnp.pad(jnp.asarray(b1, jnp.float32).reshape(H, 1), ((0, pad_h), (0, 0)))
    w2t = jnp.pad(jnp.asarray(w2, jnp.float32).T, ((0, 0), (0, pad_h)))
    b2c = jnp.asarray(b2, jnp.float32).reshape(S, 1)

    N = y_flat.shape[1]
    Np = _ceil_to(N, tile_n)
    if Np != N:
        y_flat = jnp.pad(y_flat, ((0, 0), (0, Np - N)))

    return pl.pallas_call(
        _mlp_body,
        out_shape=jax.ShapeDtypeStruct((S, Np), jnp.float32),
        grid=(Np // tile_n,),
        in_specs=[
            pl.BlockSpec((1, tile_n), lambda i: (0, i)),
            pl.BlockSpec((_H_PAD, 1), lambda i: (0, 0)),
            pl.BlockSpec((_H_PAD, 1), lambda i: (0, 0)),
            pl.BlockSpec((S, _H_PAD), lambda i: (0, 0)),
            pl.BlockSpec((S, 1), lambda i: (0, 0)),
        ],
        out_specs=pl.BlockSpec((S, tile_n), lambda i: (0, i)),
        compiler_params=pltpu.CompilerParams(dimension_semantics=("parallel",)),
    )(y_flat, w1c, b1c, w2t, b2c)[:, :N]


def _make_viterbi_body(tile_t: int, unroll: int):
    def body(p_ref, det_ref, dt_ref, uv_ref):
        @pl.when(pl.program_id(0) == 0)
        def _init():
            uv_ref[...] = jnp.zeros_like(uv_ref)

        def step(i, carry):
            u, v = carry
            dt_ref[pl.ds(i, 1), :] = jnp.where(u <= v, 0.0, 1.0)
            pt = p_ref[:, i, :]
            u2 = jnp.minimum(u - pt[0:1], v - pt[1:2])
            v2 = jnp.minimum(u - pt[2:3], v - pt[3:4])
            return (u2, v2)

        u0 = uv_ref[0:1, :]
        v0 = uv_ref[1:2, :]
        uf, vf = lax.fori_loop(0, tile_t, step, (u0, v0), unroll=unroll)
        uv_ref[0:1, :] = uf
        uv_ref[1:2, :] = vf

        det_ref[...] = jnp.transpose(dt_ref[...], (1, 0))

    return body


def _viterbi_bits(priors_stb, *, tile_t=4096, unroll=16):
    S, T, B = priors_stb.shape
    tile_t = int(min(tile_t, _ceil_to(T, 8)))
    Tp = _ceil_to(T, tile_t)
    if Tp != T:
        priors_stb = jnp.pad(priors_stb, ((0, 0), (0, Tp - T), (0, 0)))

    det = pl.pallas_call(
        _make_viterbi_body(tile_t, int(min(unroll, tile_t))),
        out_shape=jax.ShapeDtypeStruct((B, Tp), jnp.float32),
        grid=(Tp // tile_t,),
        in_specs=[pl.BlockSpec((S, tile_t, B), lambda t: (0, t, 0))],
        out_specs=pl.BlockSpec((B, tile_t), lambda t: (0, t)),
        scratch_shapes=[
            pltpu.VMEM((tile_t, B), jnp.float32),
            pltpu.VMEM((2, B), jnp.float32),
        ],
        compiler_params=pltpu.CompilerParams(dimension_semantics=("arbitrary",)),
    )(priors_stb)
    return det[:, :T]


def kernel(y, w1, b1, w2, b2):
    B, T = y.shape
    S = w2.shape[1]
    y_flat = y.T.reshape(1, T * B).astype(jnp.float32)
    priors = _priors_flat(y_flat, w1, b1, w2, b2)
    return _viterbi_bits(priors.reshape(S, T, B))


# speedup vs baseline: 1.1110x; 1.0066x over previous
import numpy as np
import jax
import jax.numpy as jnp
from jax import lax
from jax.experimental import pallas as pl
from jax.experimental.pallas import tpu as pltpu

_H_PAD = 80


def _ceil_to(x: int, m: int) -> int:
    return ((x + m - 1) // m) * m


def _mlp_body(y_ref, w1_ref, b1_ref, w2_ref, b2_ref, o_ref):
    h = jnp.maximum(w1_ref[...] * y_ref[...] + b1_ref[...], 0.0)
    o_ref[...] = (
        jnp.dot(w2_ref[...], h, preferred_element_type=jnp.float32) + b2_ref[...]
    )


def _priors_flat(y_flat, w1, b1, w2, b2, *, tile_n=65536):
    H = w1.shape[1]
    S = w2.shape[1]
    pad_h = _H_PAD - H
    w1c = jnp.pad(jnp.asarray(w1, jnp.float32).reshape(H, 1), ((0, pad_h), (0, 0)))
    b1c = jnp.pad(jnp.asarray(b1, jnp.float32).reshape(H, 1), ((0, pad_h), (0, 0)))
    w2t = jnp.pad(jnp.asarray(w2, jnp.float32).T, ((0, 0), (0, pad_h)))
    b2c = jnp.asarray(b2, jnp.float32).reshape(S, 1)

    N = y_flat.shape[1]
    Np = _ceil_to(N, tile_n)
    if Np != N:
        y_flat = jnp.pad(y_flat, ((0, 0), (0, Np - N)))

    return pl.pallas_call(
        _mlp_body,
        out_shape=jax.ShapeDtypeStruct((S, Np), jnp.float32),
        grid=(Np // tile_n,),
        in_specs=[
            pl.BlockSpec((1, tile_n), lambda i: (0, i)),
            pl.BlockSpec((_H_PAD, 1), lambda i: (0, 0)),
            pl.BlockSpec((_H_PAD, 1), lambda i: (0, 0)),
            pl.BlockSpec((S, _H_PAD), lambda i: (0, 0)),
            pl.BlockSpec((S, 1), lambda i: (0, 0)),
        ],
        out_specs=pl.BlockSpec((S, tile_n), lambda i: (0, i)),
        compiler_params=pltpu.CompilerParams(dimension_semantics=("parallel",)),
    )(y_flat, w1c, b1c, w2t, b2c)[:, :N]


def _make_viterbi_body(tile_t: int, unroll: int):
    def body(p_ref, det_ref, dt_ref, uv_ref):
        @pl.when(pl.program_id(0) == 0)
        def _init():
            uv_ref[...] = jnp.zeros_like(uv_ref)

        def step(i, carry):
            u, v = carry
            dt_ref[pl.ds(i, 1), :] = jnp.where(u <= v, 0.0, 1.0)
            pt = p_ref[:, i, :]
            u2 = jnp.minimum(u - pt[0:1], v - pt[1:2])
            v2 = jnp.minimum(u - pt[2:3], v - pt[3:4])
            return (u2, v2)

        u0 = uv_ref[0:1, :]
        v0 = uv_ref[1:2, :]
        uf, vf = lax.fori_loop(0, tile_t, step, (u0, v0), unroll=unroll)
        uv_ref[0:1, :] = uf
        uv_ref[1:2, :] = vf

        det_ref[...] = jnp.transpose(dt_ref[...], (1, 0))

    return body


def _viterbi_bits(priors_stb, *, tile_t=4096, unroll=32):
    S, T, B = priors_stb.shape
    tile_t = int(min(tile_t, _ceil_to(T, 8)))
    Tp = _ceil_to(T, tile_t)
    if Tp != T:
        priors_stb = jnp.pad(priors_stb, ((0, 0), (0, Tp - T), (0, 0)))

    det = pl.pallas_call(
        _make_viterbi_body(tile_t, int(min(unroll, tile_t))),
        out_shape=jax.ShapeDtypeStruct((B, Tp), jnp.float32),
        grid=(Tp // tile_t,),
        in_specs=[pl.BlockSpec((S, tile_t, B), lambda t: (0, t, 0))],
        out_specs=pl.BlockSpec((B, tile_t), lambda t: (0, t)),
        scratch_shapes=[
            pltpu.VMEM((tile_t, B), jnp.float32),
            pltpu.VMEM((2, B), jnp.float32),
        ],
        compiler_params=pltpu.CompilerParams(dimension_semantics=("arbitrary",)),
    )(priors_stb)
    return det[:, :T]


def kernel(y, w1, b1, w2, b2):
    B, T = y.shape
    S = w2.shape[1]
    y_flat = y.T.reshape(1, T * B).astype(jnp.float32)
    priors = _priors_flat(y_flat, w1, b1, w2, b2)
    return _viterbi_bits(priors.reshape(S, T, B))
